# baseline (speedup 1.0000x reference)
import functools

import jax
import jax.numpy as jnp
from jax import lax
from jax.experimental import pallas as pl
from jax.experimental.pallas import tpu as pltpu
from jax.experimental.pallas import tpu_sc as plsc

BATCH = 16384
CODES = 8192
DIM = 32

BM = 1024
HALF = CODES // 2

NC = 2
NS = 16
NW = NC * NS
B_PER_W = BATCH // NW


def _half_argmin(d):
    lmin = jnp.min(d, axis=1, keepdims=True)
    ids = lax.broadcasted_iota(jnp.int32, d.shape, 1).astype(jnp.float32)
    larg = jnp.min(jnp.where(d == lmin, ids, jnp.float32(1e9)),
                   axis=1, keepdims=True)
    return lmin, larg


def _argmin_body(x_ref, w_ref, out_ref):
    xb = x_ref[...]
    l2 = jnp.sum(xb * xb, axis=1, keepdims=True)
    xm2 = xb * (-2.0)
    wall = w_ref[...]
    cl2 = lax.dot_general(xm2, wall, (((1,), (1,)), ((), ())),
                          preferred_element_type=jnp.float32)
    c2 = jnp.sum(wall * wall, axis=1)[None, :]
    d = (l2 + cl2) + c2
    v0, i0 = _half_argmin(d[:, 0:HALF])
    v1, i1 = _half_argmin(d[:, HALF:CODES])
    v0r = v0.astype(jnp.bfloat16).astype(jnp.float32)
    out_ref[...] = jnp.where(v1 < v0r, i1 + float(HALF), i0).astype(jnp.int32)


def _closest_indices(x, W):
    return pl.pallas_call(
        _argmin_body,
        grid=(BATCH // BM,),
        in_specs=[
            pl.BlockSpec((BM, DIM), lambda i: (i, 0)),
            pl.BlockSpec((CODES, DIM), lambda i: (0, 0)),
        ],
        out_specs=pl.BlockSpec((BM, 1), lambda i: (i, 0)),
        out_shape=jax.ShapeDtypeStruct((BATCH, 1), jnp.int32),
    )(x, W)


@functools.cache
def _make_sc_gather():
    @functools.partial(
        pl.kernel,
        mesh=plsc.VectorSubcoreMesh(core_axis_name="c", subcore_axis_name="s"),
        out_type=jax.ShapeDtypeStruct((BATCH, DIM), jnp.float32),
        scratch_types=[
            pltpu.VMEM((B_PER_W,), jnp.int32),
            pltpu.VMEM((B_PER_W, DIM), jnp.float32),
            pltpu.SemaphoreType.DMA,
        ],
        compiler_params=pltpu.CompilerParams(use_tc_tiling_on_sc=False),
    )
    def _sc_gather(table_hbm, idx_hbm, out_hbm, idx_v, rows_v, sem):
        wid = lax.axis_index("s") * NC + lax.axis_index("c")
        base = wid * B_PER_W
        pltpu.sync_copy(idx_hbm.at[pl.ds(base, B_PER_W)], idx_v)
        pltpu.async_copy(table_hbm.at[idx_v], rows_v, sem).wait()
        pltpu.sync_copy(rows_v, out_hbm.at[pl.ds(base, B_PER_W)])

    return _sc_gather


def kernel(x, W):
    closest = _closest_indices(x, W).reshape(BATCH)
    return _make_sc_gather()(W, closest)
